# baseline (speedup 1.0000x reference)
import functools
import math

import jax
import jax.numpy as jnp
from jax.experimental import pallas as pl
from jax.experimental.pallas import tpu as pltpu
from jax.experimental.pallas import tpu_sc as plsc

_T = 512
_NKP = 1024
_LANES = 16
_DKP = 128


def _qproj_kernel(hs_ref, wq_ref, keysp_ref, relp_ref, q_ref, scores_ref,
                  logrel_ref, qsum_scr, *, n_rows, nt, nk):
    i = pl.program_id(0)
    q = jax.lax.dot_general(
        hs_ref[...], wq_ref[...], (((1,), (1,)), ((), ())),
        preferred_element_type=jnp.float32)
    q_ref[...] = q

    @pl.when(i == 0)
    def _init():
        qsum_scr[...] = jnp.zeros_like(qsum_scr)

    qsum_scr[...] += jnp.sum(q, axis=0, keepdims=True)

    @pl.when(i == nt - 1)
    def _scores():
        sc = jax.lax.dot_general(
            qsum_scr[...], keysp_ref[...], (((1,), (1,)), ((), ())),
            preferred_element_type=jnp.float32) * (1.0 / n_rows)
        lane = jax.lax.broadcasted_iota(jnp.int32, (1, _NKP), 1)
        lr = jnp.where(lane < nk, jnp.log(jnp.clip(relp_ref[...], 1e-10)),
                       -1e30)
        logrel_ref[...] = lr
        scores_ref[...] = jnp.where(lane < nk, sc + lr, -1e30)


def _sc_route(scores_hbm, logrel_hbm, keys_hbm, values_hbm,
              ksel_hbm, vsel_hbm, relc_hbm,
              sc_v, lr_v, idx_v, ksel_v, vsel_v, relc_v, sem):
    cid = jax.lax.axis_index("c")
    sid = jax.lax.axis_index("s")

    @pl.when((cid == 0) & (sid == 0))
    def _body():
        pltpu.sync_copy(scores_hbm, sc_v)
        pltpu.sync_copy(logrel_hbm, lr_v)
        lanes = jax.lax.iota(jnp.int32, 16)
        ch = _NKP // _LANES
        idx_mat = jnp.zeros((16,), jnp.int32)
        idx_rel = jnp.full((16,), _NKP - 1, jnp.int32)
        for k in range(8):
            runmax = jax.lax.fori_loop(
                0, ch,
                lambda c, rm: jnp.maximum(rm, sc_v[pl.ds(c * 16, 16)]),
                jnp.full((16,), -3.0e38, jnp.float32))
            m = jnp.max(runmax)
            runidx = jax.lax.fori_loop(
                0, ch,
                lambda c, ri: jnp.minimum(
                    ri, jnp.where(sc_v[pl.ds(c * 16, 16)] == m,
                                  lanes + c * 16, _NKP)),
                jnp.full((16,), _NKP, jnp.int32))
            idx = jnp.min(runidx)
            sel = lanes == k
            idx_mat = jnp.where(sel, idx, idx_mat)
            idx_rel = jnp.where(sel, idx, idx_rel)

            def _mask(c, carry):
                sc_v[pl.ds(c * 16, 16)] = jnp.where(
                    lanes + c * 16 == idx, -3.0e38, sc_v[pl.ds(c * 16, 16)])
                return carry

            jax.lax.fori_loop(0, ch, _mask, 0)
        relc_v[...] = jnp.exp(plsc.load_gather(lr_v, [idx_rel]))
        idx_v[...] = idx_mat
        pltpu.async_copy(keys_hbm.at[idx_v], ksel_v, sem).wait()
        pltpu.async_copy(values_hbm.at[idx_v], vsel_v, sem).wait()
        pltpu.sync_copy(ksel_v, ksel_hbm)
        pltpu.sync_copy(vsel_v, vsel_hbm)
        pltpu.sync_copy(relc_v, relc_hbm)


def _attn_kernel(q_ref, ksel_ref, relc_ref, vsel_ref, wdown_ref, wup_ref,
                 out_ref, vd_scr, *, scale):
    i = pl.program_id(0)

    @pl.when(i == 0)
    def _vd():
        vd_scr[...] = jax.lax.dot_general(
            vsel_ref[...], wdown_ref[...], (((1,), (1,)), ((), ())),
            preferred_element_type=jnp.float32)

    relc = relc_ref[...]
    s = jax.lax.dot_general(
        q_ref[...], ksel_ref[...], (((1,), (1,)), ((), ())),
        preferred_element_type=jnp.float32) * scale
    s = jnp.where(relc > 0.0, s, -1e30)
    m = jnp.max(s, axis=-1, keepdims=True)
    e = jnp.exp(s - m) * relc
    denom = jnp.sum(e, axis=-1, keepdims=True)
    u = jax.lax.dot_general(
        e, vd_scr[...], (((1,), (0,)), ((), ())),
        preferred_element_type=jnp.float32)
    mid = u / denom
    g = mid * 0.5 * (1.0 + jax.lax.erf(mid * (1.0 / math.sqrt(2.0))))
    out_ref[...] = jax.lax.dot_general(
        g, wup_ref[...], (((1,), (1,)), ((), ())),
        preferred_element_type=jnp.float32)


def kernel(hidden_states, keys, values, reliability, Wq, Wdown, Wup):
    b, s, h = hidden_states.shape
    n = b * s
    nk, dk = keys.shape
    dv = Wdown.shape[0]
    hs = hidden_states.reshape(n, h)
    nt = n // _T
    wqp = jnp.pad(Wq, ((0, _DKP - dk), (0, 0)))
    keysp = jnp.pad(keys, ((0, _NKP - nk), (0, _DKP - dk)))
    keysw = jnp.pad(keys, ((0, 0), (0, _DKP - dk)))
    relp = jnp.pad(reliability, (0, _NKP - nk)).reshape(1, _NKP)

    q, scores, logrel = pl.pallas_call(
        functools.partial(_qproj_kernel, n_rows=n, nt=nt, nk=nk),
        grid=(nt,),
        in_specs=[
            pl.BlockSpec((_T, h), lambda i: (i, 0)),
            pl.BlockSpec((_DKP, h), lambda i: (0, 0)),
            pl.BlockSpec((_NKP, _DKP), lambda i: (0, 0)),
            pl.BlockSpec((1, _NKP), lambda i: (0, 0)),
        ],
        out_specs=[
            pl.BlockSpec((_T, _DKP), lambda i: (i, 0)),
            pl.BlockSpec((1, _NKP), lambda i: (0, 0)),
            pl.BlockSpec((1, _NKP), lambda i: (0, 0)),
        ],
        out_shape=[
            jax.ShapeDtypeStruct((n, _DKP), jnp.float32),
            jax.ShapeDtypeStruct((1, _NKP), jnp.float32),
            jax.ShapeDtypeStruct((1, _NKP), jnp.float32),
        ],
        scratch_shapes=[pltpu.VMEM((1, _DKP), jnp.float32)],
    )(hs, wqp, keysp, relp)

    ksel, vsel, relc = pl.kernel(
        _sc_route,
        out_type=[
            jax.ShapeDtypeStruct((_LANES, _DKP), jnp.float32),
            jax.ShapeDtypeStruct((_LANES, h), jnp.float32),
            jax.ShapeDtypeStruct((_LANES,), jnp.float32),
        ],
        mesh=plsc.VectorSubcoreMesh(core_axis_name="c", subcore_axis_name="s"),
        scratch_types=[
            pltpu.VMEM((_NKP,), jnp.float32),
            pltpu.VMEM((_NKP,), jnp.float32),
            pltpu.VMEM((_LANES,), jnp.int32),
            pltpu.VMEM((_LANES, _DKP), jnp.float32),
            pltpu.VMEM((_LANES, h), jnp.float32),
            pltpu.VMEM((_LANES,), jnp.float32),
            pltpu.SemaphoreType.DMA,
        ],
        compiler_params=pltpu.CompilerParams(needs_layout_passes=False),
    )(scores.reshape(_NKP), logrel.reshape(_NKP), keysw, values)

    out = pl.pallas_call(
        functools.partial(_attn_kernel, scale=1.0 / math.sqrt(dk)),
        grid=(nt,),
        in_specs=[
            pl.BlockSpec((_T, _DKP), lambda i: (i, 0)),
            pl.BlockSpec((_LANES, _DKP), lambda i: (0, 0)),
            pl.BlockSpec((1, _LANES), lambda i: (0, 0)),
            pl.BlockSpec((_LANES, h), lambda i: (0, 0)),
            pl.BlockSpec((dv, h), lambda i: (0, 0)),
            pl.BlockSpec((h, dv), lambda i: (0, 0)),
        ],
        out_specs=pl.BlockSpec((_T, h), lambda i: (i, 0)),
        out_shape=jax.ShapeDtypeStruct((n, h), jnp.float32),
        scratch_shapes=[pltpu.VMEM((_LANES, dv), jnp.float32)],
    )(q, ksel, relc.reshape(1, _LANES), vsel, Wdown, Wup)

    return out.reshape(b, s, h)


# speedup vs baseline: 1.3980x; 1.3980x over previous
import functools
import math

import jax
import jax.numpy as jnp
from jax.experimental import pallas as pl
from jax.experimental.pallas import tpu as pltpu

_TOPK = 8
_T = 512


def _fused_kernel(hs_ref, wq_ref, keys_ref, rel_ref, wdown_ref, wup_ref,
                  values_hbm, out_ref,
                  q_scr, qsum_scr, ksel_scr, relc_scr, vd_scr, vsel_scr, sem,
                  *, n_rows, scale):
    p = pl.program_id(0)
    i = pl.program_id(1)
    nk = keys_ref.shape[0]

    @pl.when(p == 0)
    def _qproj():
        q = jax.lax.dot_general(
            hs_ref[...], wq_ref[...], (((1,), (1,)), ((), ())),
            preferred_element_type=jnp.float32)
        q_scr[pl.ds(i * _T, _T), :] = q

        @pl.when(i == 0)
        def _init():
            qsum_scr[...] = jnp.zeros_like(qsum_scr)

        qsum_scr[...] += jnp.sum(q, axis=0, keepdims=True)

    @pl.when((p == 1) & (i == 0))
    def _route():
        scores = jax.lax.dot_general(
            qsum_scr[...], keys_ref[...], (((1,), (1,)), ((), ())),
            preferred_element_type=jnp.float32) * (1.0 / n_rows)
        scores = scores + jnp.log(jnp.clip(rel_ref[...], 1e-10))
        iota = jax.lax.broadcasted_iota(jnp.int32, (1, nk), 1)
        masks = []
        for _ in range(_TOPK):
            m = jnp.max(scores, axis=1, keepdims=True)
            idxv = jnp.min(jnp.where(scores == m, iota, nk), axis=1,
                           keepdims=True)
            mj = iota == idxv
            masks.append(mj)
            scores = jnp.where(mj, -jnp.inf, scores)
        onehot = jnp.concatenate(
            [mj.astype(jnp.float32) for mj in masks], axis=0)
        ksel_scr[...] = jax.lax.dot_general(
            onehot, keys_ref[...], (((1,), (0,)), ((), ())),
            preferred_element_type=jnp.float32)
        relc_scr[...] = jnp.clip(
            jax.lax.dot_general(
                onehot, rel_ref[...], (((1,), (1,)), ((), ())),
                preferred_element_type=jnp.float32), 1e-10, None)
        copies = []
        for j, mj in enumerate(masks):
            idx = jnp.min(jnp.where(mj, iota, nk))
            cp = pltpu.make_async_copy(
                values_hbm.at[pl.ds(idx, 1), :], vsel_scr.at[pl.ds(j, 1), :],
                sem)
            cp.start()
            copies.append(cp)
        for cp in copies:
            cp.wait()
        vd = jax.lax.dot_general(
            vsel_scr[...], wdown_ref[...], (((1,), (1,)), ((), ())),
            preferred_element_type=jnp.float32)
        vd_scr[...] = vd * relc_scr[...]

    @pl.when(p == 1)
    def _attn():
        q = q_scr[pl.ds(i * _T, _T), :]
        s = jax.lax.dot_general(
            q, ksel_scr[...], (((1,), (1,)), ((), ())),
            preferred_element_type=jnp.float32) * scale
        m = jnp.max(s, axis=-1, keepdims=True)
        e = jnp.exp(s - m)
        denom = jax.lax.dot_general(
            e, relc_scr[...], (((1,), (0,)), ((), ())),
            preferred_element_type=jnp.float32)
        u = jax.lax.dot_general(
            e, vd_scr[...], (((1,), (0,)), ((), ())),
            preferred_element_type=jnp.float32)
        mid = u / denom
        g = mid * 0.5 * (1.0 + jax.lax.erf(mid * (1.0 / math.sqrt(2.0))))
        out_ref[...] = jax.lax.dot_general(
            g, wup_ref[...], (((1,), (1,)), ((), ())),
            preferred_element_type=jnp.float32)


def kernel(hidden_states, keys, values, reliability, Wq, Wdown, Wup):
    b, s, h = hidden_states.shape
    n = b * s
    nk, dk = keys.shape
    dv = Wdown.shape[0]
    hs = hidden_states.reshape(n, h)
    rel_row = reliability.reshape(1, nk)
    nt = n // _T

    out = pl.pallas_call(
        functools.partial(_fused_kernel, n_rows=n, scale=1.0 / math.sqrt(dk)),
        grid=(2, nt),
        in_specs=[
            pl.BlockSpec((_T, h), lambda p, i: (jnp.where(p == 0, i, nt - 1), 0)),
            pl.BlockSpec((dk, h), lambda p, i: (0, 0)),
            pl.BlockSpec((nk, dk), lambda p, i: (0, 0)),
            pl.BlockSpec((1, nk), lambda p, i: (0, 0)),
            pl.BlockSpec((dv, h), lambda p, i: (0, 0)),
            pl.BlockSpec((h, dv), lambda p, i: (0, 0)),
            pl.BlockSpec(memory_space=pl.ANY),
        ],
        out_specs=pl.BlockSpec((_T, h), lambda p, i: (jnp.where(p == 0, 0, i), 0)),
        out_shape=jax.ShapeDtypeStruct((n, h), jnp.float32),
        scratch_shapes=[
            pltpu.VMEM((n, dk), jnp.float32),
            pltpu.VMEM((1, dk), jnp.float32),
            pltpu.VMEM((_TOPK, dk), jnp.float32),
            pltpu.VMEM((_TOPK, 1), jnp.float32),
            pltpu.VMEM((_TOPK, dv), jnp.float32),
            pltpu.VMEM((_TOPK, h), jnp.float32),
            pltpu.SemaphoreType.DMA,
        ],
        compiler_params=pltpu.CompilerParams(
            vmem_limit_bytes=63 * 1024 * 1024),
    )(hs, Wq, keys, rel_row, Wdown, Wup, values)

    return out.reshape(b, s, h)
